# baseline (speedup 1.0000x reference)
import jax
import jax.numpy as jnp
from jax.experimental import pallas as pl
from jax.experimental.pallas import tpu as pltpu

N_NODES = 100
FEAT = 32
HIDDEN = 256
EMB = 64
GNN_HID = 256
GNN_OUT = 128
FLAT_DIM = 128
FLAT_OUT = 64
T = 200
E = 3200

N_P = 112
N_H = N_P // 2

_NT = (((1,), (1,)), ((), ()))


def _fused_body(nfa_ref, nfb_ref, flat_ref, dst_ref, src_ref, wihT_ref,
                whhT_ref, bias_ref, emb_ref, ws1_ref, wn1_ref, b1_ref,
                ws2_ref, wn2_ref, b2_ref, fw_ref, fb_ref, ow_ref, ob_ref,
                out_ref):
    f32 = jnp.float32
    bf16 = jnp.bfloat16

    wihT = wihT_ref[...]
    whhT = whhT_ref[...]
    bias = bias_ref[...]

    def gru_half(x_t, h):
        gi = jnp.dot(x_t, wihT, preferred_element_type=f32) + bias
        gh = jnp.dot(h.astype(bf16), whhT, preferred_element_type=f32)
        r = jax.nn.sigmoid(gi[:, :HIDDEN] + gh[:, :HIDDEN])
        z = jax.nn.sigmoid(gi[:, HIDDEN:2 * HIDDEN] + gh[:, HIDDEN:2 * HIDDEN])
        n = jnp.tanh(gi[:, 2 * HIDDEN:] + r * gh[:, 2 * HIDDEN:])
        return n + z * (h - n)

    def step(t, carry):
        ha, hb = carry
        ha = gru_half(nfa_ref[t], ha)
        hb = gru_half(nfb_ref[t], hb)
        return ha, hb

    h0 = jnp.zeros((N_H, HIDDEN), f32)
    ha, hb = jax.lax.fori_loop(0, T, step, (h0, h0))
    h_last = jnp.concatenate([ha, hb], axis=0)

    dst = dst_ref[...]
    src = src_ref[...]
    node_iota = jax.lax.broadcasted_iota(jnp.int32, (N_P, E), 0)
    od = jnp.where(dst == node_iota, 1.0, 0.0).astype(f32)
    os_ = jnp.where(src == node_iota, 1.0, 0.0).astype(f32)
    m = jax.lax.dot_general(od, os_, _NT, preferred_element_type=f32)
    cnt = jnp.sum(od, axis=1, keepdims=True)
    inv_cnt = 1.0 / jnp.maximum(cnt, 1.0)

    emb = emb_ref[...]
    gnn_in = jnp.concatenate([h_last, emb], axis=1)
    mean1 = jnp.dot(m, gnn_in, preferred_element_type=f32) * inv_cnt
    h1 = jnp.dot(gnn_in, ws1_ref[...], preferred_element_type=f32)
    h1 = h1 + jnp.dot(mean1, wn1_ref[...], preferred_element_type=f32)
    h1 = jax.nn.relu(h1 + b1_ref[...])

    mean2 = jnp.dot(m, h1, preferred_element_type=f32) * inv_cnt
    h2 = jnp.dot(h1, ws2_ref[...], preferred_element_type=f32)
    h2 = h2 + jnp.dot(mean2, wn2_ref[...], preferred_element_type=f32)
    h2 = h2 + b2_ref[...]

    xflat = jnp.dot(flat_ref[...], fw_ref[...], preferred_element_type=f32) + fb_ref[...]
    xcat = jnp.concatenate([h2, xflat, h_last], axis=1)
    out = jnp.dot(xcat, ow_ref[...], preferred_element_type=f32) + ob_ref[...]
    out_ref[...] = out


def kernel(node_feat, flat, edge_index, W_ih, W_hh, b_ih, b_hh, emb_weight,
           W_self1, W_neigh1, b1, W_self2, W_neigh2, b2, flat_W, flat_b,
           out_W, out_b):
    f32 = jnp.float32
    bf16 = jnp.bfloat16
    pad_n = ((0, N_P - N_NODES), (0, 0))
    nf = jnp.pad(node_feat, ((0, 0), (0, N_P - N_NODES), (0, 0))).astype(bf16)
    nfa = nf[:, :N_H]
    nfb = nf[:, N_H:]
    flat_p = jnp.pad(flat, pad_n)
    emb_p = jnp.pad(emb_weight, pad_n)
    dst = edge_index[1].reshape(1, E)
    src = edge_index[0].reshape(1, E)

    out = pl.pallas_call(
        _fused_body,
        out_shape=jax.ShapeDtypeStruct((N_P, 1), f32),
    )(
        nfa, nfb, flat_p, dst, src,
        W_ih.T.astype(bf16), W_hh.T.astype(bf16),
        (b_ih + b_hh).reshape(1, -1),
        emb_p, W_self1, W_neigh1, b1.reshape(1, -1),
        W_self2, W_neigh2, b2.reshape(1, -1),
        flat_W, flat_b.reshape(1, -1), out_W, out_b.reshape(1, -1),
    )
    return out[:N_NODES, 0]


# speedup vs baseline: 1.1526x; 1.1526x over previous
import jax
import jax.numpy as jnp
from jax.experimental import pallas as pl
from jax.experimental.pallas import tpu as pltpu

N_NODES = 100
FEAT = 32
HIDDEN = 256
EMB = 64
GNN_HID = 256
GNN_OUT = 128
FLAT_DIM = 128
FLAT_OUT = 64
T = 200
E = 3200

N_P = 112
N_H = N_P // 2

_NT = (((1,), (1,)), ((), ()))


def _fused_body(nfa_ref, nfb_ref, flat_ref, dst_ref, src_ref, wihT_ref,
                whhT_ref, bias_ref, emb_ref, ws1_ref, wn1_ref, b1_ref,
                ws2_ref, wn2_ref, b2_ref, fw_ref, fb_ref, ow_ref, ob_ref,
                out_ref):
    f32 = jnp.float32
    bf16 = jnp.bfloat16

    wihT = wihT_ref[...]
    whhT = whhT_ref[...]
    bias = bias_ref[...]

    def gates(gi, gh, h):
        r = jax.nn.sigmoid(gi[:, :HIDDEN] + gh[:, :HIDDEN])
        z = jax.nn.sigmoid(gi[:, HIDDEN:2 * HIDDEN] + gh[:, HIDDEN:2 * HIDDEN])
        n = jnp.tanh(gi[:, 2 * HIDDEN:] + r * gh[:, 2 * HIDDEN:])
        return n + z * (h - n)

    def substep(t, ha, hb):
        gia = jnp.dot(nfa_ref[t], wihT, preferred_element_type=f32) + bias
        gha = jnp.dot(ha.astype(bf16), whhT, preferred_element_type=f32)
        gib = jnp.dot(nfb_ref[t], wihT, preferred_element_type=f32) + bias
        ghb = jnp.dot(hb.astype(bf16), whhT, preferred_element_type=f32)
        return gates(gia, gha, ha), gates(gib, ghb, hb)

    def step(i, carry):
        ha, hb = carry
        t = i * 2
        ha, hb = substep(t, ha, hb)
        ha, hb = substep(t + 1, ha, hb)
        return ha, hb

    h0 = jnp.zeros((N_H, HIDDEN), f32)
    ha, hb = jax.lax.fori_loop(0, T // 2, step, (h0, h0))
    h_last = jnp.concatenate([ha, hb], axis=0)

    dst = dst_ref[...]
    src = src_ref[...]
    node_iota = jax.lax.broadcasted_iota(jnp.int32, (N_P, E), 0)
    od = jnp.where(dst == node_iota, 1.0, 0.0).astype(f32)
    os_ = jnp.where(src == node_iota, 1.0, 0.0).astype(f32)
    m = jax.lax.dot_general(od, os_, _NT, preferred_element_type=f32)
    cnt = jnp.sum(od, axis=1, keepdims=True)
    inv_cnt = 1.0 / jnp.maximum(cnt, 1.0)

    emb = emb_ref[...]
    gnn_in = jnp.concatenate([h_last, emb], axis=1)
    mean1 = jnp.dot(m, gnn_in, preferred_element_type=f32) * inv_cnt
    h1 = jnp.dot(gnn_in, ws1_ref[...], preferred_element_type=f32)
    h1 = h1 + jnp.dot(mean1, wn1_ref[...], preferred_element_type=f32)
    h1 = jax.nn.relu(h1 + b1_ref[...])

    mean2 = jnp.dot(m, h1, preferred_element_type=f32) * inv_cnt
    h2 = jnp.dot(h1, ws2_ref[...], preferred_element_type=f32)
    h2 = h2 + jnp.dot(mean2, wn2_ref[...], preferred_element_type=f32)
    h2 = h2 + b2_ref[...]

    xflat = jnp.dot(flat_ref[...], fw_ref[...], preferred_element_type=f32) + fb_ref[...]
    xcat = jnp.concatenate([h2, xflat, h_last], axis=1)
    out = jnp.dot(xcat, ow_ref[...], preferred_element_type=f32) + ob_ref[...]
    out_ref[...] = out


def kernel(node_feat, flat, edge_index, W_ih, W_hh, b_ih, b_hh, emb_weight,
           W_self1, W_neigh1, b1, W_self2, W_neigh2, b2, flat_W, flat_b,
           out_W, out_b):
    f32 = jnp.float32
    bf16 = jnp.bfloat16
    pad_n = ((0, N_P - N_NODES), (0, 0))
    nf = jnp.pad(node_feat, ((0, 0), (0, N_P - N_NODES), (0, 0))).astype(bf16)
    nfa = nf[:, :N_H]
    nfb = nf[:, N_H:]
    flat_p = jnp.pad(flat, pad_n)
    emb_p = jnp.pad(emb_weight, pad_n)
    dst = edge_index[1].reshape(1, E)
    src = edge_index[0].reshape(1, E)

    out = pl.pallas_call(
        _fused_body,
        out_shape=jax.ShapeDtypeStruct((N_P, 1), f32),
    )(
        nfa, nfb, flat_p, dst, src,
        W_ih.T.astype(bf16), W_hh.T.astype(bf16),
        (b_ih + b_hh).reshape(1, -1),
        emb_p, W_self1, W_neigh1, b1.reshape(1, -1),
        W_self2, W_neigh2, b2.reshape(1, -1),
        flat_W, flat_b.reshape(1, -1), out_W, out_b.reshape(1, -1),
    )
    return out[:N_NODES, 0]
